# baseline (speedup 1.0000x reference)
import functools

import jax
import jax.numpy as jnp
from jax import lax
from jax.experimental import pallas as pl
from jax.experimental.pallas import tpu as pltpu
from jax.experimental.pallas import tpu_sc as plsc

VOCAB_SIZE = 100000
D_MODEL = 768
CTX_LENGTH = 2048
BATCH = 4

LANES = 16
KG = D_MODEL // LANES

_info = plsc.get_sparse_core_info()
NC = _info.num_cores
NS = _info.num_subcores
NW = NC * NS
LBLK = CTX_LENGTH // NW
LSUB = 8
CROWS = LSUB * BATCH
NCH = LBLK // LSUB
NBUF = 3


def _emb_kernel(tok_hbm, table_hbm, pos_hbm, out_hbm,
                idx_v, pos_v, e0, e1, e2,
                gs0, gs1, gs2, ss0, ss1, ss2, psem, isem):
    wid = lax.axis_index("s") * NC + lax.axis_index("c")
    lbase = wid * LBLK
    embufs = [e0, e1, e2]
    gsems = [gs0, gs1, gs2]
    ssems = [ss0, ss1, ss2]

    pcp = pltpu.async_copy(pos_hbm.at[0, pl.ds(lbase, LBLK)], pos_v, psem)

    idx_cps = {}

    def load_idx(j):
        idx_cps[j] = [pltpu.async_copy(
            tok_hbm.at[b, pl.ds(lbase + j * LSUB, LSUB)],
            idx_v.at[j, 0, pl.ds(b * LSUB, LSUB)], isem)
            for b in range(BATCH)]

    gathers = {}
    scatters = {}

    def start_gather(c):
        cb = c % NBUF
        for cp in idx_cps[c]:
            cp.wait()
        gathers[c] = pltpu.async_copy(
            table_hbm.at[idx_v.at[c, 0]], embufs[cb], gsems[cb])

    load_idx(0)
    load_idx(1)
    start_gather(0)
    start_gather(1)
    for j in range(2, NCH):
        load_idx(j)
    pcp.wait()

    for c in range(NCH):
        cb = c % NBUF
        e = embufs[cb]
        gathers[c].wait()

        @plsc.parallel_loop(0, KG)
        def col_body(k):
            for dl in range(LSUB):
                pv = pos_v[c * LSUB + dl, pl.ds(k * LANES, LANES)]
                for b in range(BATCH):
                    plsc.addupdate(
                        e.at[b * LSUB + dl, pl.ds(k * LANES, LANES)], pv)

        if c + 2 < NCH:
            if c - 1 >= 0:
                for cp in scatters[c - 1]:
                    cp.wait()
            start_gather(c + 2)

        scatters[c] = []
        for b in range(BATCH):
            scatters[c].append(pltpu.async_copy(
                e.at[pl.ds(b * LSUB, LSUB)],
                out_hbm.at[b, pl.ds(lbase + c * LSUB, LSUB)], ssems[cb]))

    for c in (NCH - 3, NCH - 2, NCH - 1):
        for cp in scatters[c]:
            cp.wait()


@jax.jit
def _run(tokens, table, pos):
    mesh = plsc.VectorSubcoreMesh(core_axis_name="c", subcore_axis_name="s")
    k = functools.partial(
        pl.kernel,
        mesh=mesh,
        out_type=jax.ShapeDtypeStruct((BATCH, CTX_LENGTH, D_MODEL), jnp.float32),
        scratch_types=[
            pltpu.VMEM((NCH, 1, CROWS), jnp.int32),
            pltpu.VMEM((LBLK, D_MODEL), jnp.float32),
            pltpu.VMEM((CROWS, D_MODEL), jnp.float32),
            pltpu.VMEM((CROWS, D_MODEL), jnp.float32),
            pltpu.VMEM((CROWS, D_MODEL), jnp.float32),
            pltpu.SemaphoreType.DMA,
            pltpu.SemaphoreType.DMA,
            pltpu.SemaphoreType.DMA,
            pltpu.SemaphoreType.DMA,
            pltpu.SemaphoreType.DMA,
            pltpu.SemaphoreType.DMA,
            pltpu.SemaphoreType.DMA,
            pltpu.SemaphoreType.DMA,
        ],
    )(_emb_kernel)
    return k(tokens, table, pos)


def kernel(tokens, token_embedding, positional_encoding):
    return _run(tokens.astype(jnp.int32), token_embedding, positional_encoding)
